# baseline (speedup 1.0000x reference)
import functools

import jax
import jax.numpy as jnp
from jax import lax
from jax.experimental import pallas as pl
from jax.experimental.pallas import tpu as pltpu
from jax.experimental.pallas import tpu_sc as plsc

N = 10000
E = 320000
VOCAB = 64
EMBED = 128
HIDDEN = 256
NCLS = 10
NGRAPH = 128

NC = 2
NS = 16
L = 16
CH = 128

EP = 327680
NCHUNKS = EP // CH
ROWS_PER_TILE = N // NS
ACC_ROWS = N + 8
OHW = 128


def _zero_vmem_2d(buf, rows, width):
  zeros = jnp.zeros((L,), jnp.float32)

  def body(r, carry):
    for j in range(width // L):
      buf[r, pl.ds(j * L, L)] = zeros
    return carry

  lax.fori_loop(0, rows, body, 0)


def _sc_pipe_body(width, per_core_split, table_hbm, src_hbm, dst_hbm, out_hbm,
                  srcb, dstb, rowbuf, acc,
                  isem0, isem1, isem2, isem3, gsem0, gsem1, ssem0, ssem1):
  c = lax.axis_index("c")
  s = lax.axis_index("s")
  isems = (isem0, isem1, isem2, isem3)
  gsems = (gsem0, gsem1)
  ssems = (ssem0, ssem1)

  _zero_vmem_2d(rowbuf.at[0], CH, width)
  row0 = s * ROWS_PER_TILE
  for j in range(4):
    pltpu.sync_copy(rowbuf.at[0], acc.at[pl.ds(row0 + j * CH, CH)])
  pltpu.sync_copy(rowbuf.at[0, pl.ds(0, ROWS_PER_TILE - 4 * CH)],
                  acc.at[pl.ds(row0 + 4 * CH, ROWS_PER_TILE - 4 * CH)])

  plsc.subcore_barrier()

  if per_core_split:
    n = NCHUNKS // NC // NS
    chunk0 = c * (NCHUNKS // NC) + s * n
    src_off = 0
  else:
    n = NCHUNKS // NS
    chunk0 = s * n
    src_off = c * EP

  def issue_idx(ck, islot):
    eb = (chunk0 + ck) * CH
    pltpu.async_copy(src_hbm.at[pl.ds(src_off + eb, CH)], srcb.at[islot],
                     isems[islot])
    pltpu.async_copy(dst_hbm.at[pl.ds(eb, CH)], dstb.at[islot], isems[islot])

  def wait_idx(islot):
    pltpu.make_async_copy(src_hbm.at[pl.ds(0, CH)], srcb.at[islot],
                          isems[islot]).wait()
    pltpu.make_async_copy(dst_hbm.at[pl.ds(0, CH)], dstb.at[islot],
                          isems[islot]).wait()

  def issue_gather(islot, rslot):
    pltpu.async_copy(table_hbm.at[srcb.at[islot]], rowbuf.at[rslot],
                     gsems[rslot])

  def wait_gather(rslot):
    pltpu.make_async_copy(table_hbm.at[srcb.at[0]], rowbuf.at[rslot],
                          gsems[rslot]).wait()

  def issue_scatter(islot, rslot):
    pltpu.async_copy(rowbuf.at[rslot], acc.at[dstb.at[islot]], ssems[rslot],
                     add=True)

  def wait_scatter(rslot):
    pltpu.make_async_copy(rowbuf.at[rslot], acc.at[dstb.at[0]],
                          ssems[rslot]).wait()

  def step(ck, islot, first=False, noidx=False, nogather=False):
    rslot = islot % 2
    wait_gather(rslot)
    issue_scatter(islot, rslot)
    if not first:
      wait_scatter(1 - rslot)
    if not noidx:
      issue_idx(ck + 3, (islot + 3) % 4)
    if not nogather:
      wait_idx((islot + 1) % 4)
      issue_gather((islot + 1) % 4, 1 - rslot)

  issue_idx(0, 0)
  issue_idx(1, 1)
  issue_idx(2, 2)
  wait_idx(0)
  issue_gather(0, 0)

  step(0, 0, first=True)
  step(1, 1)
  step(2, 2)
  step(3, 3)

  nq = n // 4

  def quad(q, carry):
    c0 = 4 * q
    step(c0 + 0, 0)
    step(c0 + 1, 1)
    step(c0 + 2, 2)
    step(c0 + 3, 3)
    return carry

  lax.fori_loop(1, nq - 1, quad, 0)

  step(n - 4, 0)
  step(n - 3, 1, noidx=True)
  step(n - 2, 2, noidx=True)
  step(n - 1, 3, noidx=True, nogather=True)
  wait_scatter(1)

  plsc.subcore_barrier()

  pltpu.sync_copy(acc.at[pl.ds(row0, ROWS_PER_TILE)], out_hbm.at[c * NS + s])


def _make_sc_pipe(width, per_core_split):
  mesh = plsc.VectorSubcoreMesh(core_axis_name="c", subcore_axis_name="s")
  body = functools.partial(_sc_pipe_body, width, per_core_split)
  return pl.kernel(
      body,
      out_type=jax.ShapeDtypeStruct((NC * NS, ROWS_PER_TILE, width),
                                    jnp.float32),
      mesh=mesh,
      compiler_params=pltpu.CompilerParams(needs_layout_passes=False),
      scratch_types=[
          pltpu.VMEM((4, CH), jnp.int32),
          pltpu.VMEM((4, CH), jnp.int32),
          pltpu.VMEM((2, CH, width), jnp.float32),
          pltpu.VMEM_SHARED((ACC_ROWS, width), jnp.float32),
          pltpu.SemaphoreType.DMA,
          pltpu.SemaphoreType.DMA,
          pltpu.SemaphoreType.DMA,
          pltpu.SemaphoreType.DMA,
          pltpu.SemaphoreType.DMA,
          pltpu.SemaphoreType.DMA,
          pltpu.SemaphoreType.DMA,
          pltpu.SemaphoreType.DMA,
      ],
  )


def _tc_oh_body(xi_ref, oh_ref):
  oh_ref[...] = (xi_ref[...] ==
                 lax.broadcasted_iota(jnp.int32, (ROWB_OH, OHW), 1)
                 ).astype(jnp.float32)


ROWB_OH = 2000


def _tc_onehot(xi2d):
  return pl.pallas_call(
      _tc_oh_body,
      grid=(N // ROWB_OH,),
      in_specs=[pl.BlockSpec((ROWB_OH, 1), lambda i: (i, 0))],
      out_specs=pl.BlockSpec((ROWB_OH, OHW), lambda i: (i, 0)),
      out_shape=jax.ShapeDtypeStruct((N, OHW), jnp.float32),
  )(xi2d)


ROWB = 400
NROWB = N // ROWB


def _dotT(a, b):
  return lax.dot_general(a, b, (((1,), (1,)), ((), ())),
                         preferred_element_type=jnp.float32)


def _tc1_body(c2_ref, xi_ref, emb_ref, wl1_ref, bl1_ref, wr1_ref,
              wl2_ref, wr2_ref, bl2_ref, m2_ref, r2_ref, deg_ref):
  cb = (c2_ref[0] + c2_ref[1])[:, :VOCAB]
  deg = jnp.sum(cb, axis=1, keepdims=True)
  degc = jnp.maximum(deg, 1.0)
  cn = cb / degc

  b1 = _dotT(emb_ref[...], wl1_ref[...])
  b1r = _dotT(emb_ref[...], wr1_ref[...])

  onehot = (xi_ref[...] ==
            lax.broadcasted_iota(jnp.int32, (ROWB, VOCAB), 1)
            ).astype(jnp.float32)

  h1 = jnp.maximum(
      jnp.dot(cn, b1, preferred_element_type=jnp.float32)
      + jnp.dot(onehot, b1r, preferred_element_type=jnp.float32)
      + bl1_ref[...], 0.0)

  m2 = _dotT(h1, wl2_ref[...])
  r2 = _dotT(h1, wr2_ref[...]) + bl2_ref[...]

  m2_ref[0] = m2[:, :EMBED]
  m2_ref[1] = m2[:, EMBED:]
  r2_ref[...] = r2
  deg_ref[...] = deg


def _tc_layer1(c2, xi2d, emb, wl1, bl1, wr1, wl2, wr2, bl2):
  full = lambda shape: pl.BlockSpec(shape, lambda i: (0,) * len(shape))
  return pl.pallas_call(
      _tc1_body,
      grid=(NROWB,),
      in_specs=[
          pl.BlockSpec((NC, ROWB, OHW), lambda i: (0, i, 0)),
          pl.BlockSpec((ROWB, 1), lambda i: (i, 0)),
          full((VOCAB, EMBED)),
          full((HIDDEN, EMBED)),
          full((1, HIDDEN)),
          full((HIDDEN, EMBED)),
          full((HIDDEN, HIDDEN)),
          full((HIDDEN, HIDDEN)),
          full((1, HIDDEN)),
      ],
      out_specs=[
          pl.BlockSpec((NC, ROWB, EMBED), lambda i: (0, i, 0)),
          pl.BlockSpec((ROWB, HIDDEN), lambda i: (i, 0)),
          pl.BlockSpec((ROWB, 1), lambda i: (i, 0)),
      ],
      out_shape=[
          jax.ShapeDtypeStruct((NC, N, EMBED), jnp.float32),
          jax.ShapeDtypeStruct((N, HIDDEN), jnp.float32),
          jax.ShapeDtypeStruct((N, 1), jnp.float32),
      ],
  )(c2, xi2d, emb, wl1, bl1, wr1, wl2, wr2, bl2)


def _tc2_body(s2_ref, r2_ref, deg_ref, batch_ref, wlin_ref, blin_ref,
              out_ref, acc, cnt):
  i = pl.program_id(0)

  @pl.when(i == 0)
  def _init():
    acc[...] = jnp.zeros_like(acc)
    cnt[...] = jnp.zeros_like(cnt)

  s2 = jnp.concatenate([s2_ref[0], s2_ref[1]], axis=1)
  degc = jnp.maximum(deg_ref[...], 1.0)
  h2 = jnp.maximum(s2 / degc + r2_ref[...], 0.0)

  onehot_t = (jnp.reshape(batch_ref[...], (1, ROWB)) ==
              lax.broadcasted_iota(jnp.int32, (NGRAPH, ROWB), 0)
              ).astype(jnp.float32)

  acc[...] += jnp.dot(onehot_t, h2, preferred_element_type=jnp.float32)
  cnt[...] += jnp.sum(onehot_t, axis=1, keepdims=True)

  @pl.when(i == NROWB - 1)
  def _finish():
    pooled = acc[...] / jnp.maximum(cnt[...], 1.0)
    out_ref[...] = _dotT(pooled, wlin_ref[...]) + blin_ref[...]


def _tc_layer2(s2, r2, deg, batch2d, wlin, blin):
  full = lambda shape: pl.BlockSpec(shape, lambda i: (0,) * len(shape))
  return pl.pallas_call(
      _tc2_body,
      grid=(NROWB,),
      in_specs=[
          pl.BlockSpec((NC, ROWB, EMBED), lambda i: (0, i, 0)),
          pl.BlockSpec((ROWB, HIDDEN), lambda i: (i, 0)),
          pl.BlockSpec((ROWB, 1), lambda i: (i, 0)),
          pl.BlockSpec((ROWB, 1), lambda i: (i, 0)),
          full((NCLS, HIDDEN)),
          full((1, NCLS)),
      ],
      out_specs=pl.BlockSpec((NGRAPH, NCLS), lambda i: (0, 0)),
      out_shape=jax.ShapeDtypeStruct((NGRAPH, NCLS), jnp.float32),
      scratch_shapes=[
          pltpu.VMEM((NGRAPH, HIDDEN), jnp.float32),
          pltpu.VMEM((NGRAPH, 1), jnp.float32),
      ],
  )(s2, r2, deg, batch2d, wlin, blin)


@jax.jit
def kernel(x, edge_index, batch, embed_table, Wl1, bl1, Wr1, Wl2, bl2, Wr2,
           Wlin, blin):
  src = jnp.concatenate(
      [edge_index[0], jnp.zeros((EP - E,), edge_index.dtype)])
  dst = jnp.concatenate(
      [edge_index[1], jnp.full((EP - E,), N, edge_index.dtype)])
  srcs = jnp.concatenate([src, src + N])

  c2 = _make_sc_pipe(OHW, True)(_tc_onehot(x), src, dst)

  m2, r2, deg = _tc_layer1(
      jnp.reshape(c2, (NC, N, OHW)),
      jnp.reshape(x, (N, 1)),
      embed_table, Wl1, jnp.reshape(bl1, (1, HIDDEN)), Wr1,
      Wl2, Wr2, jnp.reshape(bl2, (1, HIDDEN)))

  s2 = _make_sc_pipe(EMBED, False)(
      jnp.reshape(m2, (NC * N, EMBED)), srcs, dst)

  return _tc_layer2(
      jnp.reshape(s2, (NC, N, EMBED)), r2, deg,
      jnp.reshape(batch, (N, 1)), Wlin, jnp.reshape(blin, (1, NCLS)))


# speedup vs baseline: 1.8628x; 1.8628x over previous
import functools

import jax
import jax.numpy as jnp
from jax import lax
from jax.experimental import pallas as pl
from jax.experimental.pallas import tpu as pltpu
from jax.experimental.pallas import tpu_sc as plsc

N = 10000
E = 320000
VOCAB = 64
EMBED = 128
HIDDEN = 256
NCLS = 10
NGRAPH = 128

NC = 2
NS = 16
L = 16
CH = 128

NCHUNKS = E // CH
ROWS_PER_TILE = N // NS
K = 2
OHW = 128


def _zero_vmem_2d(buf, rows, width):
  zeros = jnp.zeros((L,), jnp.float32)

  def body(r, carry):
    for j in range(width // L):
      buf[r, pl.ds(j * L, L)] = zeros
    return carry

  lax.fori_loop(0, rows, body, 0)


def _zero_acc_slice(rowbuf, acc, row0):
  _zero_vmem_2d(rowbuf.at[0], CH, rowbuf.shape[2])
  nfull = ROWS_PER_TILE // CH
  for j in range(nfull):
    pltpu.sync_copy(rowbuf.at[0], acc.at[pl.ds(row0 + j * CH, CH)])
  rem = ROWS_PER_TILE - nfull * CH
  pltpu.sync_copy(rowbuf.at[0, pl.ds(0, rem)],
                  acc.at[pl.ds(row0 + nfull * CH, rem)])


def _load_batch(src_hbm, dst_hbm, idxb, isem, p, chunk0, src_off, g):
  for b in range(K):
    eb = (chunk0 + g * K + b) * CH
    pltpu.async_copy(src_hbm.at[pl.ds(src_off + eb, CH)], idxb.at[p * K + b],
                     isem)
    pltpu.async_copy(dst_hbm.at[pl.ds(eb, CH)], idxb.at[2 * K + p * K + b],
                     isem)


def _wait_batch_loads(src_hbm, dst_hbm, idxb, isem, p):
  for b in range(K):
    pltpu.make_async_copy(src_hbm.at[pl.ds(0, CH)], idxb.at[p * K + b],
                          isem).wait()
    pltpu.make_async_copy(dst_hbm.at[pl.ds(0, CH)],
                          idxb.at[2 * K + p * K + b], isem).wait()


def _gs_batch(table_hbm, idxb, rowbuf, acc, gsem, ssem, p):
  gds = [pltpu.async_copy(table_hbm.at[idxb.at[p * K + b]], rowbuf.at[b],
                          gsem)
         for b in range(K)]
  for d in gds:
    d.wait()
  sds = [pltpu.async_copy(rowbuf.at[b], acc.at[idxb.at[2 * K + p * K + b]],
                          ssem, add=True)
         for b in range(K)]
  for d in sds:
    d.wait()


def _tail_chunk(table_hbm, src_hbm, dst_hbm, idxb, rowbuf, acc,
                isem, gsem, ssem, ck, src_off):
  eb = ck * CH
  d1 = pltpu.async_copy(src_hbm.at[pl.ds(src_off + eb, CH)], idxb.at[0], isem)
  d2 = pltpu.async_copy(dst_hbm.at[pl.ds(eb, CH)], idxb.at[2 * K], isem)
  d1.wait()
  d2.wait()
  g = pltpu.async_copy(table_hbm.at[idxb.at[0]], rowbuf.at[0], gsem)
  g.wait()
  sd = pltpu.async_copy(rowbuf.at[0], acc.at[idxb.at[2 * K]], ssem, add=True)
  sd.wait()


def _sc_body(per_core_split, width, table_hbm, src_hbm, dst_hbm, out_hbm,
             idxb, rowbuf, acc, isemA, isemB, gsem, ssem):
  c = lax.axis_index("c")
  s = lax.axis_index("s")
  isems = (isemA, isemB)

  row0 = s * ROWS_PER_TILE
  _zero_acc_slice(rowbuf, acc, row0)

  plsc.subcore_barrier()

  if per_core_split:
    chunk0 = c * (NCHUNKS // NC) + s * 78 + jnp.minimum(s, 2)
    src_off = 0
    nbatches = 78 // K
    tail_cond = s < 2
    tail_ck = chunk0 + 78
  else:
    chunk0 = s * 156 + jnp.minimum(s, 4)
    src_off = c * E
    nbatches = 156 // K
    tail_cond = s < 4
    tail_ck = chunk0 + 156

  load = functools.partial(_load_batch, src_hbm, dst_hbm, idxb)
  wait_loads = functools.partial(_wait_batch_loads, src_hbm, dst_hbm, idxb)
  gs = functools.partial(_gs_batch, table_hbm, idxb, rowbuf, acc, gsem, ssem)

  load(isems[0], 0, chunk0, src_off, 0)

  def pair(q, carry):
    g0 = 2 * q
    load(isems[1], 1, chunk0, src_off, g0 + 1)
    wait_loads(isems[0], 0)
    gs(0)
    load(isems[0], 0, chunk0, src_off, g0 + 2)
    wait_loads(isems[1], 1)
    gs(1)
    return carry

  if nbatches % 2 == 0:
    lax.fori_loop(0, nbatches // 2 - 1, pair, 0)
    load(isems[1], 1, chunk0, src_off, nbatches - 1)
    wait_loads(isems[0], 0)
    gs(0)
    wait_loads(isems[1], 1)
    gs(1)
  else:
    lax.fori_loop(0, nbatches // 2, pair, 0)
    wait_loads(isems[0], 0)
    gs(0)

  @pl.when(tail_cond)
  def _tail():
    _tail_chunk(table_hbm, src_hbm, dst_hbm, idxb, rowbuf, acc,
                isemA, gsem, ssem, tail_ck, src_off)

  plsc.subcore_barrier()

  pltpu.sync_copy(acc.at[pl.ds(row0, ROWS_PER_TILE)], out_hbm.at[c * NS + s])


def _make_sc_pipe(width, per_core_split):
  mesh = plsc.VectorSubcoreMesh(core_axis_name="c", subcore_axis_name="s")
  body = functools.partial(_sc_body, per_core_split, width)
  return pl.kernel(
      body,
      out_type=jax.ShapeDtypeStruct((NC * NS, ROWS_PER_TILE, width),
                                    jnp.float32),
      mesh=mesh,
      compiler_params=pltpu.CompilerParams(needs_layout_passes=False),
      scratch_types=[
          pltpu.VMEM((4 * K, CH), jnp.int32),
          pltpu.VMEM((K, CH, width), jnp.float32),
          pltpu.VMEM_SHARED((N, width), jnp.float32),
          pltpu.SemaphoreType.DMA,
          pltpu.SemaphoreType.DMA,
          pltpu.SemaphoreType.DMA,
          pltpu.SemaphoreType.DMA,
      ],
  )


def _tc_oh_body(xi_ref, oh_ref):
  oh_ref[...] = (xi_ref[...] ==
                 lax.broadcasted_iota(jnp.int32, (ROWB_OH, OHW), 1)
                 ).astype(jnp.float32)


ROWB_OH = 2000


def _tc_onehot(xi2d):
  return pl.pallas_call(
      _tc_oh_body,
      grid=(N // ROWB_OH,),
      in_specs=[pl.BlockSpec((ROWB_OH, 1), lambda i: (i, 0))],
      out_specs=pl.BlockSpec((ROWB_OH, OHW), lambda i: (i, 0)),
      out_shape=jax.ShapeDtypeStruct((N, OHW), jnp.float32),
  )(xi2d)


ROWB = 400
NROWB = N // ROWB


def _dotT(a, b):
  return lax.dot_general(a, b, (((1,), (1,)), ((), ())),
                         preferred_element_type=jnp.float32)


def _tc1_body(c2_ref, xi_ref, emb_ref, wl1_ref, bl1_ref, wr1_ref,
              wl2_ref, wr2_ref, bl2_ref, m2_ref, r2_ref, deg_ref):
  cb = (c2_ref[0] + c2_ref[1])[:, :VOCAB]
  deg = jnp.sum(cb, axis=1, keepdims=True)
  degc = jnp.maximum(deg, 1.0)
  cn = cb / degc

  b1 = _dotT(emb_ref[...], wl1_ref[...])
  b1r = _dotT(emb_ref[...], wr1_ref[...])

  onehot = (xi_ref[...] ==
            lax.broadcasted_iota(jnp.int32, (ROWB, VOCAB), 1)
            ).astype(jnp.float32)

  h1 = jnp.maximum(
      jnp.dot(cn, b1, preferred_element_type=jnp.float32)
      + jnp.dot(onehot, b1r, preferred_element_type=jnp.float32)
      + bl1_ref[...], 0.0)

  m2 = _dotT(h1, wl2_ref[...])
  r2 = _dotT(h1, wr2_ref[...]) + bl2_ref[...]

  m2_ref[0] = m2[:, :EMBED]
  m2_ref[1] = m2[:, EMBED:]
  r2_ref[...] = r2
  deg_ref[...] = deg


def _tc_layer1(c2, xi2d, emb, wl1, bl1, wr1, wl2, wr2, bl2):
  full = lambda shape: pl.BlockSpec(shape, lambda i: (0,) * len(shape))
  return pl.pallas_call(
      _tc1_body,
      grid=(NROWB,),
      in_specs=[
          pl.BlockSpec((NC, ROWB, OHW), lambda i: (0, i, 0)),
          pl.BlockSpec((ROWB, 1), lambda i: (i, 0)),
          full((VOCAB, EMBED)),
          full((HIDDEN, EMBED)),
          full((1, HIDDEN)),
          full((HIDDEN, EMBED)),
          full((HIDDEN, HIDDEN)),
          full((HIDDEN, HIDDEN)),
          full((1, HIDDEN)),
      ],
      out_specs=[
          pl.BlockSpec((NC, ROWB, EMBED), lambda i: (0, i, 0)),
          pl.BlockSpec((ROWB, HIDDEN), lambda i: (i, 0)),
          pl.BlockSpec((ROWB, 1), lambda i: (i, 0)),
      ],
      out_shape=[
          jax.ShapeDtypeStruct((NC, N, EMBED), jnp.float32),
          jax.ShapeDtypeStruct((N, HIDDEN), jnp.float32),
          jax.ShapeDtypeStruct((N, 1), jnp.float32),
      ],
  )(c2, xi2d, emb, wl1, bl1, wr1, wl2, wr2, bl2)


def _tc2_body(s2_ref, r2_ref, deg_ref, batch_ref, wlin_ref, blin_ref,
              out_ref, acc, cnt):
  i = pl.program_id(0)

  @pl.when(i == 0)
  def _init():
    acc[...] = jnp.zeros_like(acc)
    cnt[...] = jnp.zeros_like(cnt)

  s2 = jnp.concatenate([s2_ref[0], s2_ref[1]], axis=1)
  degc = jnp.maximum(deg_ref[...], 1.0)
  h2 = jnp.maximum(s2 / degc + r2_ref[...], 0.0)

  onehot_t = (jnp.reshape(batch_ref[...], (1, ROWB)) ==
              lax.broadcasted_iota(jnp.int32, (NGRAPH, ROWB), 0)
              ).astype(jnp.float32)

  acc[...] += jnp.dot(onehot_t, h2, preferred_element_type=jnp.float32)
  cnt[...] += jnp.sum(onehot_t, axis=1, keepdims=True)

  @pl.when(i == NROWB - 1)
  def _finish():
    pooled = acc[...] / jnp.maximum(cnt[...], 1.0)
    out_ref[...] = _dotT(pooled, wlin_ref[...]) + blin_ref[...]


def _tc_layer2(s2, r2, deg, batch2d, wlin, blin):
  full = lambda shape: pl.BlockSpec(shape, lambda i: (0,) * len(shape))
  return pl.pallas_call(
      _tc2_body,
      grid=(NROWB,),
      in_specs=[
          pl.BlockSpec((NC, ROWB, EMBED), lambda i: (0, i, 0)),
          pl.BlockSpec((ROWB, HIDDEN), lambda i: (i, 0)),
          pl.BlockSpec((ROWB, 1), lambda i: (i, 0)),
          pl.BlockSpec((ROWB, 1), lambda i: (i, 0)),
          full((NCLS, HIDDEN)),
          full((1, NCLS)),
      ],
      out_specs=pl.BlockSpec((NGRAPH, NCLS), lambda i: (0, 0)),
      out_shape=jax.ShapeDtypeStruct((NGRAPH, NCLS), jnp.float32),
      scratch_shapes=[
          pltpu.VMEM((NGRAPH, HIDDEN), jnp.float32),
          pltpu.VMEM((NGRAPH, 1), jnp.float32),
      ],
  )(s2, r2, deg, batch2d, wlin, blin)


@jax.jit
def kernel(x, edge_index, batch, embed_table, Wl1, bl1, Wr1, Wl2, bl2, Wr2,
           Wlin, blin):
  src = edge_index[0]
  dst = edge_index[1]
  srcs = jnp.concatenate([src, src + N])

  c2 = _make_sc_pipe(OHW, True)(_tc_onehot(x), src, dst)

  m2, r2, deg = _tc_layer1(
      jnp.reshape(c2, (NC, N, OHW)),
      jnp.reshape(x, (N, 1)),
      embed_table, Wl1, jnp.reshape(bl1, (1, HIDDEN)), Wr1,
      Wl2, Wr2, jnp.reshape(bl2, (1, HIDDEN)))

  s2 = _make_sc_pipe(EMBED, False)(
      jnp.reshape(m2, (NC * N, EMBED)), srcs, dst)

  return _tc_layer2(
      jnp.reshape(s2, (NC, N, EMBED)), r2, deg,
      jnp.reshape(batch, (N, 1)), Wlin, jnp.reshape(blin, (1, NCLS)))
